# Initial kernel scaffold; baseline (speedup 1.0000x reference)
import jax
import jax.numpy as jnp
from jax.experimental import pallas as pl


def kernel(feats, edge_index, s):
    raise NotImplementedError("write your pallas kernel here")


# speedup vs baseline: 4.0050x; 4.0050x over previous
import functools

import jax
import jax.numpy as jnp
from jax import lax
from jax.experimental import pallas as pl
from jax.experimental.pallas import tpu as pltpu
from jax.experimental.pallas import tpu_sc as plsc

_N = 10000
_E = 320000
_D = 128
_K = 10

_NC = 2
_NS = 16
_NW = _NC * _NS

_C = 128
_KCH = -(-_E // (_NW * _C))
_E_PAD = _NW * _KCH * _C
_N_PAD = 10240
_RPT = _N_PAD // _NS
_ZB = 64

_mesh = plsc.VectorSubcoreMesh(core_axis_name="c", subcore_axis_name="s")


@functools.partial(
    pl.kernel,
    mesh=_mesh,
    out_type=jax.ShapeDtypeStruct((_NC, _N_PAD, 16), jnp.float32),
    scratch_types=[
        pltpu.VMEM((_KCH, _C), jnp.int32),
        pltpu.VMEM((_C + _ZB, 16), jnp.float32),
        pltpu.VMEM_SHARED((_N_PAD, 16), jnp.float32),
    ],
)
def _deg(dst_hbm, const_hbm, out_hbm, didx_v, const_v, acc_sh):
    cid = lax.axis_index("c")
    sid = lax.axis_index("s")
    w = cid * _NS + sid
    pltpu.sync_copy(dst_hbm.at[w], didx_v)
    pltpu.sync_copy(const_hbm, const_v)
    base = sid * _RPT
    for i in range(_RPT // _ZB):
        pltpu.sync_copy(const_v.at[pl.ds(_C, _ZB)],
                        acc_sh.at[pl.ds(base + i * _ZB, _ZB)])
    plsc.subcore_barrier()

    def chunk(j, carry):
        pltpu.sync_copy(const_v.at[pl.ds(0, _C)],
                        acc_sh.at[didx_v.at[j]], add=True)
        return carry

    lax.fori_loop(0, _KCH, chunk, 0)
    plsc.subcore_barrier()
    pltpu.sync_copy(acc_sh.at[pl.ds(base, _RPT)],
                    out_hbm.at[cid, pl.ds(base, _RPT)])


@functools.partial(
    pl.kernel,
    mesh=_mesh,
    out_type=jax.ShapeDtypeStruct((_NC, _N_PAD, _D), jnp.float32),
    scratch_types=[
        pltpu.VMEM((_KCH, _C), jnp.int32),
        pltpu.VMEM((_KCH, _C), jnp.int32),
        pltpu.VMEM((_C, _D), jnp.float32),
        pltpu.VMEM((_ZB, _D), jnp.float32),
        pltpu.VMEM_SHARED((_N_PAD, _D), jnp.float32),
        pltpu.SemaphoreType.DMA,
    ],
)
def _hop(m_hbm, src_hbm, dst_hbm, z_hbm, out_hbm,
         sidx_v, didx_v, rows_v, zeros_v, acc_sh, sem):
    cid = lax.axis_index("c")
    sid = lax.axis_index("s")
    w = cid * _NS + sid
    pltpu.sync_copy(src_hbm.at[w], sidx_v)
    pltpu.sync_copy(dst_hbm.at[w], didx_v)
    pltpu.sync_copy(z_hbm, zeros_v)
    base = sid * _RPT
    for i in range(_RPT // _ZB):
        pltpu.sync_copy(zeros_v, acc_sh.at[pl.ds(base + i * _ZB, _ZB)])
    plsc.subcore_barrier()

    def chunk(j, carry):
        pltpu.async_copy(m_hbm.at[sidx_v.at[j]], rows_v, sem).wait()
        pltpu.sync_copy(rows_v, acc_sh.at[didx_v.at[j]], add=True)
        return carry

    lax.fori_loop(0, _KCH, chunk, 0)
    plsc.subcore_barrier()
    pltpu.sync_copy(acc_sh.at[pl.ds(base, _RPT)],
                    out_hbm.at[cid, pl.ds(base, _RPT)])


def _norm_body(degp_ref, feats_ref, norm_ref, nsq_ref, m0_ref):
    deg = degp_ref[0, :, 0:1] + degp_ref[1, :, 0:1]
    norm = lax.rsqrt(deg)
    norm_ref[...] = norm
    nsq_ref[...] = norm * norm
    m0_ref[...] = feats_ref[...] * norm


def _combine_body(p_ref, norm_ref, nsq_ref, h_ref, m_ref):
    psum = p_ref[0] + p_ref[1]
    h_ref[...] = psum * norm_ref[...]
    m_ref[...] = psum * nsq_ref[...]


def _final_body(*refs):
    s_ref = refs[0]
    h_refs = refs[1:2 + _K]
    out_ref = refs[2 + _K]
    st = s_ref[...]
    acc = jnp.zeros((_BN, _D), jnp.float32)
    for hr in h_refs:
        hv = hr[...]
        logit = jnp.sum(hv * st, axis=1, keepdims=True)
        sg = 1.0 / (1.0 + jnp.exp(-logit))
        acc = acc + sg * hv
    out_ref[...] = acc


_BN = 1024


def _norm_call(degp, feats_p):
    return pl.pallas_call(
        _norm_body,
        grid=(_N_PAD // _BN,),
        in_specs=[
            pl.BlockSpec((_NC, _BN, 16), lambda i: (0, i, 0)),
            pl.BlockSpec((_BN, _D), lambda i: (i, 0)),
        ],
        out_specs=[
            pl.BlockSpec((_BN, 1), lambda i: (i, 0)),
            pl.BlockSpec((_BN, 1), lambda i: (i, 0)),
            pl.BlockSpec((_BN, _D), lambda i: (i, 0)),
        ],
        out_shape=[
            jax.ShapeDtypeStruct((_N_PAD, 1), jnp.float32),
            jax.ShapeDtypeStruct((_N_PAD, 1), jnp.float32),
            jax.ShapeDtypeStruct((_N_PAD, _D), jnp.float32),
        ],
    )(degp, feats_p)


def _combine_call(p, norm, nsq):
    return pl.pallas_call(
        _combine_body,
        grid=(_N_PAD // _BN,),
        in_specs=[
            pl.BlockSpec((_NC, _BN, _D), lambda i: (0, i, 0)),
            pl.BlockSpec((_BN, 1), lambda i: (i, 0)),
            pl.BlockSpec((_BN, 1), lambda i: (i, 0)),
        ],
        out_specs=[
            pl.BlockSpec((_BN, _D), lambda i: (i, 0)),
            pl.BlockSpec((_BN, _D), lambda i: (i, 0)),
        ],
        out_shape=[
            jax.ShapeDtypeStruct((_N_PAD, _D), jnp.float32),
            jax.ShapeDtypeStruct((_N_PAD, _D), jnp.float32),
        ],
    )(p, norm, nsq)


def _final_call(s_t, hs):
    return pl.pallas_call(
        _final_body,
        grid=(_N_PAD // _BN,),
        in_specs=[pl.BlockSpec((1, _D), lambda i: (0, 0))]
        + [pl.BlockSpec((_BN, _D), lambda i: (i, 0)) for _ in hs],
        out_specs=pl.BlockSpec((_BN, _D), lambda i: (i, 0)),
        out_shape=jax.ShapeDtypeStruct((_N_PAD, _D), jnp.float32),
    )(s_t, *hs)


def kernel(feats, edge_index, s):
    feats = feats.astype(jnp.float32)
    src = edge_index[0].astype(jnp.int32)
    dst = edge_index[1].astype(jnp.int32)

    pad_e = _E_PAD - _E
    pad_idx = jnp.full((pad_e,), _N, jnp.int32)
    src_p = jnp.concatenate([src, pad_idx]).reshape(_NW, _KCH, _C)
    dst_p = jnp.concatenate([dst, pad_idx]).reshape(_NW, _KCH, _C)
    feats_p = jnp.pad(feats, ((0, _N_PAD - _N), (0, 0)))

    const16 = jnp.concatenate(
        [jnp.ones((_C, 16), jnp.float32), jnp.zeros((_ZB, 16), jnp.float32)])
    z128 = jnp.zeros((_ZB, _D), jnp.float32)

    degp = _deg(dst_p, const16)
    norm, nsq, m = _norm_call(degp, feats_p)

    hs = [feats_p]
    for _ in range(_K):
        p = _hop(m, src_p, dst_p, z128)
        h, m = _combine_call(p, norm, nsq)
        hs.append(h)

    out = _final_call(jnp.transpose(s), hs)
    return out[:_N]
